# baseline (speedup 1.0000x reference)
import functools

import jax
import jax.numpy as jnp
from jax import lax
from jax.experimental import pallas as pl
from jax.experimental.pallas import tpu as pltpu
from jax.experimental.pallas import tpu_sc as plsc


def kernel(x):
    B, N, S, D = x.shape
    info = plsc.get_sparse_core_info()
    NC, NS = info.num_cores, info.num_subcores
    NW = NC * NS
    pairs = B * 2
    w_per_pair = NW // pairs
    chunk = S // w_per_pair
    mesh = plsc.VectorSubcoreMesh(core_axis_name="c", subcore_axis_name="s")

    @functools.partial(
        pl.kernel,
        mesh=mesh,
        out_type=jax.ShapeDtypeStruct((B, 2, S, D), x.dtype),
    )
    def k(x_hbm, out_hbm):
        wid = lax.axis_index("s") * NC + lax.axis_index("c")
        pair = wid // w_per_pair
        slot = wid % w_per_pair
        b = pair // 2
        j = pair % 2
        off = slot * chunk
        pltpu.sync_copy(
            x_hbm.at[b, 2 * j, pl.ds(off, chunk)],
            out_hbm.at[b, j, pl.ds(off, chunk)],
        )

    return k(x)


# speedup vs baseline: 11.0070x; 11.0070x over previous
import functools

import jax
import jax.numpy as jnp
from jax import lax
from jax.experimental import pallas as pl
from jax.experimental.pallas import tpu as pltpu
from jax.experimental.pallas import tpu_sc as plsc


def kernel(x):
    B, N, S, D = x.shape
    info = plsc.get_sparse_core_info()
    NC, NS = info.num_cores, info.num_subcores
    NW = NC * NS
    pairs = B * 2
    w_per_pair = NW // pairs
    chunk = S // w_per_pair
    mesh = plsc.VectorSubcoreMesh(core_axis_name="c", subcore_axis_name="s")

    @functools.partial(
        pl.kernel,
        mesh=mesh,
        out_type=jax.ShapeDtypeStruct((B, 2, S, D), x.dtype),
        scratch_types=[pltpu.VMEM((chunk, D), x.dtype)],
    )
    def k(x_hbm, out_hbm, buf):
        wid = lax.axis_index("s") * NC + lax.axis_index("c")
        pair = wid // w_per_pair
        slot = wid % w_per_pair
        b = pair // 2
        j = pair % 2
        off = slot * chunk
        pltpu.sync_copy(x_hbm.at[b, 2 * j, pl.ds(off, chunk)], buf)
        pltpu.sync_copy(buf, out_hbm.at[b, j, pl.ds(off, chunk)])

    return k(x)
